# Initial kernel scaffold; baseline (speedup 1.0000x reference)
import jax
import jax.numpy as jnp
from jax.experimental import pallas as pl


def kernel(x, edge_index, batch, sex, mutation, age, Wl1, bl1, Wr1, br1, att1, bias1, Wl2, bl2, Wr2, br2, att2, bias2, pw1, pb1, pw2, pb2, gamma, beta, ew1, eb1, ew2, eb2):
    raise NotImplementedError("write your pallas kernel here")


# speedup vs baseline: 21.5213x; 21.5213x over previous
import functools

import jax
import jax.numpy as jnp
from jax import lax
from jax.experimental import pallas as pl
from jax.experimental.pallas import tpu as pltpu
from jax.experimental.pallas import tpu_sc as plsc

N = 10000
H, C = 4, 64
HC = H * C
L = 16
NW = 32
NPW = 320
NPAD = NW * NPW
EB = 64
PTRP = NPAD + 16


def _lane_sum(v):
    lane = lax.iota(jnp.int32, L)
    for sh in (8, 4, 2, 1):
        v = v + jnp.take_along_axis(v, lane ^ sh, axis=0,
                                    mode="promise_in_bounds")
    return v


def _sc_gat_body(xl_hbm, xr_hbm, src_hbm, dst_hbm, ptr_hbm, att_hbm, bias_hbm,
                 out_hbm, ptr_v, att_v, bias_v, srcv, dstv, dsts, xjb, xib,
                 outb, sb, sem1, sem2):
    wid = lax.axis_index("s") * 2 + lax.axis_index("c")
    base = wid * NPW

    pltpu.sync_copy(ptr_hbm.at[pl.ds(base, NPW + 16)], ptr_v)
    pltpu.sync_copy(att_hbm, att_v)
    pltpu.sync_copy(bias_hbm, bias_v)

    zero = jnp.zeros((L,), jnp.float32)

    def zbody(n, _):
        for j in range(HC // L):
            outb[n, pl.ds(L * j, L)] = zero
        sb[pl.ds(n * L, L)] = zero
        return 0

    lax.fori_loop(0, NPW, zbody, 0)

    e0 = ptr_v[pl.ds(0, L)][0]
    e1 = ptr_v[pl.ds(NPW, L)][0]
    estart = (e0 // EB) * EB
    nblk = (e1 - estart + EB - 1) // EB

    def blk(b, _):
        off = estart + b * EB
        pltpu.sync_copy(src_hbm.at[pl.ds(off, EB)], srcv)
        pltpu.sync_copy(dst_hbm.at[pl.ds(off, EB)], dstv)
        pltpu.sync_copy(dst_hbm.at[pl.ds(off, EB)], dsts.at[pl.ds(0, EB)])
        cp1 = pltpu.async_copy(xl_hbm.at[srcv], xjb, sem1)
        cp2 = pltpu.async_copy(xr_hbm.at[dstv], xib, sem2)
        cp1.wait()
        cp2.wait()

        def edge(e, _):
            eg = off + e

            @pl.when(jnp.logical_and(eg >= e0, eg < e1))
            def _():
                nloc = dsts[pl.ds(e, L)][0] - base
                xjs = [xjb[e, pl.ds(L * j, L)] for j in range(HC // L)]
                ws = []
                for j in range(HC // L):
                    t = xib[e, pl.ds(L * j, L)] + xjs[j]
                    lr = jnp.where(t >= 0.0, t, t * 0.2)
                    ws.append(lr * att_v[pl.ds(L * j, L)])
                ps = []
                for h in range(H):
                    hs = (ws[4 * h] + ws[4 * h + 1]) + (ws[4 * h + 2] + ws[4 * h + 3])
                    ps.append(jnp.exp(_lane_sum(hs)))
                for j in range(HC // L):
                    plsc.addupdate(outb.at[nloc, pl.ds(L * j, L)], ps[j // 4] * xjs[j])
                lane = lax.iota(jnp.int32, L)
                psel = jnp.where(lane < 4, ps[0],
                                 jnp.where(lane < 8, ps[1],
                                           jnp.where(lane < 12, ps[2], ps[3])))
                plsc.addupdate(sb.at[pl.ds(nloc * L, L)], psel)

            return 0

        lax.fori_loop(0, EB, edge, 0)
        return 0

    lax.fori_loop(0, nblk, blk, 0)

    def fin(n, _):
        srow = sb[pl.ds(n * L, L)]
        for h in range(H):
            inv = 1.0 / (jnp.full((L,), srow[4 * h], jnp.float32) + 1e-16)
            for j in range(4):
                c = 4 * h + j
                t = outb[n, pl.ds(L * c, L)] * inv + bias_v[pl.ds(L * c, L)]
                outb[n, pl.ds(L * c, L)] = jnp.where(t > 0.0, t, jnp.exp(t) - 1.0)
        return 0

    lax.fori_loop(0, NPW, fin, 0)
    pltpu.sync_copy(outb, out_hbm.at[pl.ds(base, NPW)])


_sc_gat = functools.partial(
    pl.kernel,
    out_type=jax.ShapeDtypeStruct((NPAD, HC), jnp.float32),
    mesh=plsc.VectorSubcoreMesh(core_axis_name="c", subcore_axis_name="s",
                                num_cores=2, num_subcores=16),
    scratch_types=[
        pltpu.VMEM((NPW + 16,), jnp.int32),
        pltpu.VMEM((HC,), jnp.float32),
        pltpu.VMEM((HC,), jnp.float32),
        pltpu.VMEM((EB,), jnp.int32),
        pltpu.VMEM((EB,), jnp.int32),
        pltpu.VMEM((EB + L,), jnp.int32),
        pltpu.VMEM((EB, HC), jnp.float32),
        pltpu.VMEM((EB, HC), jnp.float32),
        pltpu.VMEM((NPW, HC), jnp.float32),
        pltpu.VMEM((NPW * L,), jnp.float32),
        pltpu.SemaphoreType.DMA,
        pltpu.SemaphoreType.DMA,
    ],
)(_sc_gat_body)


_RB = 1000


def _proj_body(x_ref, wl_ref, bl_ref, wr_ref, br_ref, pw_ref, xl_ref, xr_ref,
               pool_ref, *, mean_pool):
    xb = x_ref[...]
    xl_ref[...] = jnp.dot(xb, wl_ref[...],
                          preferred_element_type=jnp.float32) + bl_ref[...]
    xr_ref[...] = jnp.dot(xb, wr_ref[...],
                          preferred_element_type=jnp.float32) + br_ref[...]
    if mean_pool:
        pool_ref[...] = jnp.mean(xb, axis=1, keepdims=True)
    else:
        pool_ref[...] = jnp.dot(xb, pw_ref[...],
                                preferred_element_type=jnp.float32)


def _proj(x, wl, bl, wr, br, pw, mean_pool):
    d = x.shape[1]
    grid = (N // _RB,)
    return pl.pallas_call(
        functools.partial(_proj_body, mean_pool=mean_pool),
        grid=grid,
        in_specs=[
            pl.BlockSpec((_RB, d), lambda i: (i, 0)),
            pl.BlockSpec((d, HC), lambda i: (0, 0)),
            pl.BlockSpec((1, HC), lambda i: (0, 0)),
            pl.BlockSpec((d, HC), lambda i: (0, 0)),
            pl.BlockSpec((1, HC), lambda i: (0, 0)),
            pl.BlockSpec((d, 1), lambda i: (0, 0)),
        ],
        out_specs=[
            pl.BlockSpec((_RB, HC), lambda i: (i, 0)),
            pl.BlockSpec((_RB, HC), lambda i: (i, 0)),
            pl.BlockSpec((_RB, 1), lambda i: (i, 0)),
        ],
        out_shape=[
            jax.ShapeDtypeStruct((N, HC), jnp.float32),
            jax.ShapeDtypeStruct((N, HC), jnp.float32),
            jax.ShapeDtypeStruct((N, 1), jnp.float32),
        ],
    )(x, wl, bl.reshape(1, HC), wr, br.reshape(1, HC), pw)


def _pool_body(h_ref, pw_ref, o_ref):
    o_ref[...] = jnp.dot(h_ref[...], pw_ref[...],
                         preferred_element_type=jnp.float32)


def _pool(h, pw):
    return pl.pallas_call(
        _pool_body,
        grid=(N // _RB,),
        in_specs=[
            pl.BlockSpec((_RB, HC), lambda i: (i, 0)),
            pl.BlockSpec((HC, 1), lambda i: (0, 0)),
        ],
        out_specs=pl.BlockSpec((_RB, 1), lambda i: (i, 0)),
        out_shape=jax.ShapeDtypeStruct((N, 1), jnp.float32),
    )(h, pw)


def _head_body(xs_ref, g_ref, b_ref, w_ref, eb1_ref, ew2_ref, eb2_ref,
               lno_ref, pred_ref, acc_ref):
    i = pl.program_id(0)
    v = xs_ref[...].reshape(1, N)
    mu = jnp.mean(v)
    var = jnp.mean((v - mu) ** 2)
    ln = (v - mu) / jnp.sqrt(var + 1e-5) * g_ref[...] + b_ref[...]
    lno_ref[...] = ln.reshape(1, 1, N)
    w = w_ref[...].reshape(N, HC)
    contrib = jnp.dot(ln, w, preferred_element_type=jnp.float32)

    @pl.when(i == 0)
    def _():
        acc_ref[...] = contrib

    @pl.when(i > 0)
    def _():
        acc_ref[...] += contrib

    @pl.when(i == 2)
    def _():
        hfc = jnp.maximum(acc_ref[...] + eb1_ref[...], 0.0)
        pred_ref[...] = jnp.dot(hfc, ew2_ref[...],
                                preferred_element_type=jnp.float32) + eb2_ref[...]


def _head(xs, gamma, beta, ew1, eb1, ew2, eb2):
    return pl.pallas_call(
        _head_body,
        grid=(3,),
        in_specs=[
            pl.BlockSpec((1, 1, N), lambda i: (i, 0, 0)),
            pl.BlockSpec((1, N), lambda i: (0, 0)),
            pl.BlockSpec((1, N), lambda i: (0, 0)),
            pl.BlockSpec((1, N, HC), lambda i: (i, 0, 0)),
            pl.BlockSpec((1, HC), lambda i: (0, 0)),
            pl.BlockSpec((HC, 2), lambda i: (0, 0)),
            pl.BlockSpec((1, 2), lambda i: (0, 0)),
        ],
        out_specs=[
            pl.BlockSpec((1, 1, N), lambda i: (i, 0, 0)),
            pl.BlockSpec((1, 2), lambda i: (0, 0)),
        ],
        out_shape=[
            jax.ShapeDtypeStruct((3, 1, N), jnp.float32),
            jax.ShapeDtypeStruct((1, 2), jnp.float32),
        ],
        scratch_shapes=[pltpu.VMEM((1, HC), jnp.float32)],
    )(xs.reshape(3, 1, N), gamma.reshape(1, N), beta.reshape(1, N),
      ew1.reshape(3, N, HC), eb1.reshape(1, HC), ew2, eb2.reshape(1, 2))


def kernel(x, edge_index, batch, sex, mutation, age,
           Wl1, bl1, Wr1, br1, att1, bias1,
           Wl2, bl2, Wr2, br2, att2, bias2,
           pw1, pb1, pw2, pb2, gamma, beta,
           ew1, eb1, ew2, eb2):
    loops = jnp.arange(N, dtype=edge_index.dtype)
    src = jnp.concatenate([edge_index[0], loops])
    dst = jnp.concatenate([edge_index[1], loops])
    order = jnp.argsort(dst)
    src_s = src[order]
    dst_s = dst[order]
    ptr = jnp.searchsorted(dst_s, jnp.arange(PTRP, dtype=jnp.int32)
                           ).astype(jnp.int32)
    pad = jnp.zeros((EB + L,), jnp.int32)
    src_p = jnp.concatenate([src_s, pad])
    dst_p = jnp.concatenate([dst_s, pad])

    xl1, xr1, x0col = _proj(x, Wl1, bl1, Wr1, br1,
                            jnp.zeros((x.shape[1], 1), jnp.float32), True)
    h1 = _sc_gat(xl1, xr1, src_p, dst_p, ptr,
                 att1.reshape(HC), bias1)[:N]
    xl2, xr2, x1col = _proj(h1, Wl2, bl2, Wr2, br2, pw1, False)
    x1col = x1col + pb1
    h2 = _sc_gat(xl2, xr2, src_p, dst_p, ptr,
                 att2.reshape(HC), bias2)[:N]
    x2col = _pool(h2, pw2) + pb2

    xs = jnp.concatenate([x0col.reshape(1, N), x1col.reshape(1, N),
                          x2col.reshape(1, N)], axis=0)
    lno, pred = _head(xs, gamma, beta, ew1, eb1, ew2, eb2)
    lno = lno.reshape(3, N)
    x0 = lno[0:1]
    x1 = lno[1:2]
    x2 = lno[2:3]
    ms = lno.reshape(1, 3 * N)
    return (pred, x0, x1, x2, ms)
